# baseline (speedup 1.0000x reference)
import functools

import jax
import jax.numpy as jnp
from jax import lax
from jax.experimental import pallas as pl
from jax.experimental.pallas import tpu as pltpu
from jax.experimental.pallas import tpu_sc as plsc

_info = plsc.get_sparse_core_info()
_NC, _NS, _L = _info.num_cores, _info.num_subcores, _info.num_lanes
_NW = _NC * _NS


@functools.lru_cache(maxsize=None)
def _build(batch, seq_len, vocab, dim):
    total = batch * seq_len
    s_per_w = seq_len // _NW
    assert seq_len % _NW == 0 and s_per_w % 8 == 0 and dim % _L == 0

    half = s_per_w // 2

    def body(idx_hbm, pos_hbm, table_hbm, out_hbm, idx_v, pos_v, acc_v,
             s_pa, s_pb, s_p0, *sems):
        s_idx, s_g, s_o = sems[:batch], sems[batch:2 * batch], sems[2 * batch:]
        wid = lax.axis_index("s") * _NC + lax.axis_index("c")
        s0 = wid * s_per_w
        cp_idx = [
            pltpu.async_copy(idx_hbm.at[pl.ds(b * seq_len + s0, s_per_w)],
                             idx_v.at[b], s_idx[b])
            for b in range(batch)
        ]
        cp_p0 = pltpu.async_copy(pos_hbm.at[pl.ds(s0, s_per_w), :],
                                 acc_v.at[pl.ds(0, s_per_w), :], s_p0)
        cp_pa = pltpu.async_copy(pos_hbm.at[pl.ds(s0, half), :],
                                 pos_v.at[pl.ds(0, half), :], s_pa)
        cp_pb = pltpu.async_copy(pos_hbm.at[pl.ds(s0 + half, half), :],
                                 pos_v.at[pl.ds(half, half), :], s_pb)
        cp_g = [None] * batch
        cp_p0.wait()
        cp_idx[0].wait()
        cp_g[0] = pltpu.async_copy(
            table_hbm.at[idx_v.at[0]], acc_v.at[pl.ds(0, s_per_w), :],
            s_g[0], add=True)

        def copy_rows(dst_off, src_off, n):
            @plsc.parallel_loop(0, n, 1, unroll=4)
            def _cp(i):
                for j in range(dim // _L):
                    sl = pl.ds(j * _L, _L)
                    acc_v[dst_off + i, sl] = pos_v[src_off + i, sl]

        cp_pa.wait()
        copy_rows(s_per_w, 0, half)
        cp_pb.wait()
        copy_rows(s_per_w + half, half, half)
        cp_idx[1].wait()
        cp_g[1] = pltpu.async_copy(
            table_hbm.at[idx_v.at[1]],
            acc_v.at[pl.ds(s_per_w, s_per_w), :], s_g[1], add=True)
        for b in range(2, batch):
            copy_rows(b * s_per_w, 0, s_per_w)
            cp_idx[b].wait()
            cp_g[b] = pltpu.async_copy(
                table_hbm.at[idx_v.at[b]],
                acc_v.at[pl.ds(b * s_per_w, s_per_w), :], s_g[b], add=True)
        cp_o = []
        for b in range(batch):
            cp_g[b].wait()
            cp_o.append(pltpu.async_copy(
                acc_v.at[pl.ds(b * s_per_w, s_per_w), :],
                out_hbm.at[pl.ds(b * seq_len + s0, s_per_w), :], s_o[b]))
        for cp in cp_o:
            cp.wait()

    mesh = plsc.VectorSubcoreMesh(core_axis_name="c", subcore_axis_name="s")
    kern = pl.kernel(
        body,
        mesh=mesh,
        out_type=jax.ShapeDtypeStruct((total, dim), jnp.float32),
        scratch_types=[
            pltpu.VMEM((batch, s_per_w), jnp.int32),
            pltpu.VMEM((s_per_w, dim), jnp.float32),
            pltpu.VMEM((batch * s_per_w, dim), jnp.float32),
            pltpu.SemaphoreType.DMA,
            pltpu.SemaphoreType.DMA,
            pltpu.SemaphoreType.DMA,
        ] + [pltpu.SemaphoreType.DMA] * (3 * batch),
    )

    @jax.jit
    def run(input_ids, token_table, pos_table):
        idx_flat = input_ids.reshape(-1).astype(jnp.int32)
        out = kern(idx_flat, pos_table, token_table)
        return out.reshape(batch, seq_len, dim)

    return run


def kernel(input_ids, token_table, pos_table):
    batch, seq_len = input_ids.shape
    vocab, dim = token_table.shape
    return _build(batch, seq_len, vocab, dim)(input_ids, token_table, pos_table)


# speedup vs baseline: 1.0458x; 1.0458x over previous
import functools

import jax
import jax.numpy as jnp
from jax import lax
from jax.experimental import pallas as pl
from jax.experimental.pallas import tpu as pltpu
from jax.experimental.pallas import tpu_sc as plsc

_info = plsc.get_sparse_core_info()
_NC, _NS, _L = _info.num_cores, _info.num_subcores, _info.num_lanes
_NW = _NC * _NS


@functools.lru_cache(maxsize=None)
def _build(batch, seq_len, vocab, dim):
    s_per_w = seq_len // _NW
    assert seq_len % _NW == 0 and s_per_w % 8 == 0 and dim % _L == 0

    def body(idx_hbm, pos_hbm, table_hbm, out_hbm, idx_v, pos_v, acc_v,
             s_i, s_pos, s_p0, *sems):
        s_g, s_o = sems[:batch], sems[batch:]
        wid = lax.axis_index("s") * _NC + lax.axis_index("c")
        s0 = wid * s_per_w
        cp_idx = [
            pltpu.async_copy(idx_hbm.at[b, pl.ds(s0, s_per_w)], idx_v.at[b],
                             s_i)
            for b in range(batch)
        ]
        pos_src = pos_hbm.at[pl.ds(s0, s_per_w), :]
        cp_p0 = pltpu.async_copy(pos_src, acc_v.at[0], s_p0)
        cp_pos = pltpu.async_copy(pos_src, pos_v, s_pos)
        cp_g = [None] * batch
        cp_p0.wait()
        for cp in cp_idx:
            cp.wait()
        cp_g[0] = pltpu.async_copy(table_hbm.at[idx_v.at[0]], acc_v.at[0],
                                   s_g[0], add=True)
        cp_pos.wait()
        for b in range(1, batch):
            def row(i, carry, _b=b):
                for j in range(dim // _L):
                    sl = pl.ds(j * _L, _L)
                    acc_v[_b, i, sl] = pos_v[i, sl]
                return carry

            lax.fori_loop(0, s_per_w, row, 0)
            cp_g[b] = pltpu.async_copy(table_hbm.at[idx_v.at[b]], acc_v.at[b],
                                       s_g[b], add=True)
        cp_o = []
        for b in range(batch):
            cp_g[b].wait()
            cp_o.append(pltpu.async_copy(
                acc_v.at[b], out_hbm.at[b, pl.ds(s0, s_per_w), :], s_o[b]))
        for cp in cp_o:
            cp.wait()

    mesh = plsc.VectorSubcoreMesh(core_axis_name="c", subcore_axis_name="s")
    kern = pl.kernel(
        body,
        mesh=mesh,
        out_type=jax.ShapeDtypeStruct((batch, seq_len, dim), jnp.float32),
        scratch_types=[
            pltpu.VMEM((batch, s_per_w), jnp.int32),
            pltpu.VMEM((s_per_w, dim), jnp.float32),
            pltpu.VMEM((batch, s_per_w, dim), jnp.float32),
            pltpu.SemaphoreType.DMA,
            pltpu.SemaphoreType.DMA,
            pltpu.SemaphoreType.DMA,
        ] + [pltpu.SemaphoreType.DMA] * (2 * batch),
    )

    @jax.jit
    def run(input_ids, token_table, pos_table):
        return kern(input_ids.astype(jnp.int32), pos_table, token_table)

    return run


def kernel(input_ids, token_table, pos_table):
    batch, seq_len = input_ids.shape
    vocab, dim = token_table.shape
    return _build(batch, seq_len, vocab, dim)(input_ids, token_table, pos_table)
